# Initial kernel scaffold; baseline (speedup 1.0000x reference)
import jax
import jax.numpy as jnp
from jax.experimental import pallas as pl


def kernel(x, edge_index, W1, b1, W2, b2):
    raise NotImplementedError("write your pallas kernel here")


# speedup vs baseline: 25.4465x; 25.4465x over previous
import functools

import jax
import jax.numpy as jnp
from jax import lax
from jax.experimental import pallas as pl
from jax.experimental.pallas import tpu as pltpu
from jax.experimental.pallas import tpu_sc as plsc

N = 10000
HID = 64
NC = 2
NS = 16
L = 16
NW = NC * NS
CB = 128
N_ACC = 10240
RPT = N_ACC // NS


def _mesh():
    return plsc.VectorSubcoreMesh(core_axis_name="c", subcore_axis_name="s")


_SC_PARAMS = pltpu.CompilerParams(use_tc_tiling_on_sc=False)


def _fill(ref, rows, cols, val):
    @pl.loop(0, rows)
    def _(i):
        @pl.loop(0, cols, step=L)
        def _(j):
            ref.at[pl.ds(i, 1), pl.ds(j, L)][...] = jnp.full((1, L), val, jnp.float32)


def _sc_deg(dst3):
    chn = dst3.shape[1]

    @functools.partial(
        pl.kernel,
        out_type=jax.ShapeDtypeStruct((NC, N_ACC, L), jnp.float32),
        mesh=_mesh(),
        scratch_types=[
            pltpu.VMEM((chn, CB), jnp.int32),
            pltpu.VMEM((CB, L), jnp.float32),
            pltpu.VMEM((CB, L), jnp.float32),
            pltpu.VMEM_SHARED((N_ACC, L), jnp.float32),
        ],
        compiler_params=_SC_PARAMS,
    )
    def k(dst_hbm, out_hbm, dst_v, ones_v, zb_v, acc_sh):
        c = lax.axis_index("c")
        s = lax.axis_index("s")
        w = s * NC + c
        _fill(ones_v, CB, L, 1.0)
        _fill(zb_v, CB, L, 0.0)

        @pl.loop(0, RPT, step=CB)
        def _(r):
            pltpu.sync_copy(zb_v, acc_sh.at[pl.ds(s * RPT + r, CB)])

        plsc.subcore_barrier()
        pltpu.sync_copy(dst_hbm.at[w], dst_v)

        @pl.loop(0, chn)
        def _(j):
            pltpu.sync_copy(ones_v, acc_sh.at[dst_v.at[j]], add=True)

        plsc.subcore_barrier()
        pltpu.sync_copy(acc_sh.at[pl.ds(s * RPT, RPT)],
                        out_hbm.at[c, pl.ds(s * RPT, RPT)])

    return k(dst3)


def _sc_agg(y, src3, dst3, d):
    chn = src3.shape[1]

    @functools.partial(
        pl.kernel,
        out_type=jax.ShapeDtypeStruct((NC, N_ACC, d), jnp.float32),
        mesh=_mesh(),
        scratch_types=[
            pltpu.VMEM((chn, CB), jnp.int32),
            pltpu.VMEM((chn, CB), jnp.int32),
            pltpu.VMEM((CB, d), jnp.float32),
            pltpu.VMEM((CB, d), jnp.float32),
            pltpu.VMEM_SHARED((N_ACC, d), jnp.float32),
            pltpu.SemaphoreType.DMA,
        ],
        compiler_params=_SC_PARAMS,
    )
    def k(y_hbm, src_hbm, dst_hbm, out_hbm,
          src_v, dst_v, rows_v, zb_v, acc_sh, sem):
        c = lax.axis_index("c")
        s = lax.axis_index("s")
        w = s * NC + c
        _fill(zb_v, CB, d, 0.0)

        @pl.loop(0, RPT, step=CB)
        def _(r):
            pltpu.sync_copy(zb_v, acc_sh.at[pl.ds(s * RPT + r, CB)])

        plsc.subcore_barrier()
        pltpu.sync_copy(src_hbm.at[w], src_v)
        pltpu.sync_copy(dst_hbm.at[w], dst_v)

        @pl.loop(0, chn)
        def _(j):
            pltpu.async_copy(y_hbm.at[src_v.at[j]], rows_v, sem).wait()
            pltpu.sync_copy(rows_v, acc_sh.at[dst_v.at[j]], add=True)

        plsc.subcore_barrier()
        pltpu.sync_copy(acc_sh.at[pl.ds(s * RPT, RPT)],
                        out_hbm.at[c, pl.ds(s * RPT, RPT)])

    return k(y, src3, dst3)


def _dinv2(degp):
    deg = degp[0, :N, 0:1] + degp[1, :N, 0:1] + 1.0
    return lax.rsqrt(deg)


def _tc1(x, W1, degp):
    def body(x_ref, w_ref, degp_ref, y_ref):
        dinv = _dinv2(degp_ref[...])
        xw = jnp.dot(x_ref[...], w_ref[...], preferred_element_type=jnp.float32)
        y_ref[...] = xw * dinv

    return pl.pallas_call(
        body, out_shape=jax.ShapeDtypeStruct((N, HID), jnp.float32),
    )(x, W1, degp)


def _tc2(a1p, y1, degp, b1, W2):
    def body(a1p_ref, y1_ref, degp_ref, b1_ref, w2_ref, y2p_ref):
        dinv = _dinv2(degp_ref[...])
        a1 = a1p_ref[0, :N, :] + a1p_ref[1, :N, :]
        h = jnp.maximum(dinv * (a1 + y1_ref[...]) + b1_ref[...], 0.0)
        hw = jnp.dot(h, w2_ref[...], preferred_element_type=jnp.float32)
        y2p_ref[...] = jnp.broadcast_to(dinv * hw, (N, L))

    return pl.pallas_call(
        body, out_shape=jax.ShapeDtypeStruct((N, L), jnp.float32),
    )(a1p, y1, degp, b1, W2)


def _tc3(a2p, y2p, degp, b2):
    def body(a2p_ref, y2p_ref, degp_ref, b2_ref, o_ref):
        dinv = _dinv2(degp_ref[...])
        a2 = a2p_ref[0, :N, 0:1] + a2p_ref[1, :N, 0:1]
        y2 = y2p_ref[:, 0:1]
        o_ref[...] = jax.nn.sigmoid(dinv * (a2 + y2) + b2_ref[...])

    return pl.pallas_call(
        body, out_shape=jax.ShapeDtypeStruct((N, 1), jnp.float32),
    )(a2p, y2p, degp, b2)


def kernel(x, edge_index, W1, b1, W2, b2):
    e = edge_index.shape[1]
    grain = NW * CB
    chn = -(-e // grain)
    e_pad = chn * grain
    ei = edge_index.astype(jnp.int32)
    src = jnp.concatenate([ei[0], jnp.zeros((e_pad - e,), jnp.int32)])
    dst = jnp.concatenate([ei[1], jnp.full((e_pad - e,), N, jnp.int32)])
    src3 = src.reshape(NW, chn, CB)
    dst3 = dst.reshape(NW, chn, CB)

    degp = _sc_deg(dst3)
    y1 = _tc1(x, W1, degp)
    a1p = _sc_agg(y1, src3, dst3, HID)
    y2p = _tc2(a1p, y1, degp, b1, W2)
    a2p = _sc_agg(y2p, src3, dst3, L)
    return _tc3(a2p, y2p, degp, b2)


# speedup vs baseline: 45.7184x; 1.7966x over previous
import functools

import jax
import jax.numpy as jnp
from jax import lax
from jax.experimental import pallas as pl
from jax.experimental.pallas import tpu as pltpu
from jax.experimental.pallas import tpu_sc as plsc

N = 10000
HID = 64
NC = 2
NS = 16
L = 16
NW = NC * NS
CB = 128
N_ACC = 10240
RPT = N_ACC // NS


def _mesh():
    return plsc.VectorSubcoreMesh(core_axis_name="c", subcore_axis_name="s")


_SC_PARAMS = pltpu.CompilerParams(use_tc_tiling_on_sc=False)


def _fill(ref, rows, cols, val):
    @pl.loop(0, rows)
    def _(i):
        @pl.loop(0, cols, step=L)
        def _(j):
            ref.at[pl.ds(i, 1), pl.ds(j, L)][...] = jnp.full((1, L), val, jnp.float32)


def _sc_deg(dst3):
    chn = dst3.shape[1]

    @functools.partial(
        pl.kernel,
        out_type=jax.ShapeDtypeStruct((NC, N_ACC, L), jnp.float32),
        mesh=_mesh(),
        scratch_types=[
            pltpu.VMEM((chn, CB), jnp.int32),
            pltpu.VMEM((CB, L), jnp.float32),
            pltpu.VMEM((CB, L), jnp.float32),
            pltpu.VMEM_SHARED((N_ACC, L), jnp.float32),
        ],
        compiler_params=_SC_PARAMS,
    )
    def k(dst_hbm, out_hbm, dst_v, ones_v, zb_v, acc_sh):
        c = lax.axis_index("c")
        s = lax.axis_index("s")
        w = s * NC + c
        _fill(ones_v, CB, L, 1.0)
        _fill(zb_v, CB, L, 0.0)

        @pl.loop(0, RPT, step=CB)
        def _(r):
            pltpu.sync_copy(zb_v, acc_sh.at[pl.ds(s * RPT + r, CB)])

        plsc.subcore_barrier()
        pltpu.sync_copy(dst_hbm.at[w], dst_v)

        @pl.loop(0, chn)
        def _(j):
            pltpu.sync_copy(ones_v, acc_sh.at[dst_v.at[j]], add=True)

        plsc.subcore_barrier()
        pltpu.sync_copy(acc_sh.at[pl.ds(s * RPT, RPT)],
                        out_hbm.at[c, pl.ds(s * RPT, RPT)])

    return k(dst3)


def _sc_agg(y, src3, dst3, d):
    chn = src3.shape[1]
    npt = N // NS

    @functools.partial(
        pl.kernel,
        out_type=jax.ShapeDtypeStruct((NC, N_ACC, d), jnp.float32),
        mesh=_mesh(),
        scratch_types=[
            pltpu.VMEM((chn, CB), jnp.int32),
            pltpu.VMEM((chn, CB), jnp.int32),
            pltpu.VMEM((CB, d), jnp.float32),
            pltpu.VMEM((CB, d), jnp.float32),
            pltpu.VMEM((CB, d), jnp.float32),
            pltpu.VMEM_SHARED((N, d), jnp.float32),
            pltpu.VMEM_SHARED((N_ACC, d), jnp.float32),
            pltpu.SemaphoreType.DMA,
            pltpu.SemaphoreType.DMA,
        ],
        compiler_params=_SC_PARAMS,
    )
    def k(y_hbm, src_hbm, dst_hbm, out_hbm,
          src_v, dst_v, rows0, rows1, zb_v, y_sh, acc_sh, semg0, semg1):
        c = lax.axis_index("c")
        s = lax.axis_index("s")
        w = s * NC + c
        _fill(zb_v, CB, d, 0.0)

        @pl.loop(0, RPT, step=CB)
        def _(r):
            pltpu.sync_copy(zb_v, acc_sh.at[pl.ds(s * RPT + r, CB)])

        pltpu.sync_copy(y_hbm.at[pl.ds(s * npt, npt)], y_sh.at[pl.ds(s * npt, npt)])
        pltpu.sync_copy(src_hbm.at[w], src_v)
        pltpu.sync_copy(dst_hbm.at[w], dst_v)
        plsc.subcore_barrier()

        pltpu.async_copy(y_sh.at[src_v.at[0]], rows0, semg0)

        @pl.loop(0, chn, step=2)
        def _(j):
            a1 = pltpu.async_copy(y_sh.at[src_v.at[j + 1]], rows1, semg1)
            pltpu.make_async_copy(y_sh.at[src_v.at[j]], rows0, semg0).wait()
            pltpu.sync_copy(rows0, acc_sh.at[dst_v.at[j]], add=True)

            @pl.when(j + 2 < chn)
            def _():
                pltpu.async_copy(y_sh.at[src_v.at[j + 2]], rows0, semg0)

            a1.wait()
            pltpu.sync_copy(rows1, acc_sh.at[dst_v.at[j + 1]], add=True)

        plsc.subcore_barrier()
        pltpu.sync_copy(acc_sh.at[pl.ds(s * RPT, RPT)],
                        out_hbm.at[c, pl.ds(s * RPT, RPT)])

    return k(y, src3, dst3)


def _dinv2(degp):
    deg = degp[0, :N, 0:1] + degp[1, :N, 0:1] + 1.0
    return lax.rsqrt(deg)


def _tc1(x, W1, degp):
    def body(x_ref, w_ref, degp_ref, y_ref):
        dinv = _dinv2(degp_ref[...])
        xw = jnp.dot(x_ref[...], w_ref[...], preferred_element_type=jnp.float32)
        y_ref[...] = xw * dinv

    return pl.pallas_call(
        body, out_shape=jax.ShapeDtypeStruct((N, HID), jnp.float32),
    )(x, W1, degp)


def _tc2(a1p, y1, degp, b1, W2):
    def body(a1p_ref, y1_ref, degp_ref, b1_ref, w2_ref, y2p_ref):
        dinv = _dinv2(degp_ref[...])
        a1 = a1p_ref[0, :N, :] + a1p_ref[1, :N, :]
        h = jnp.maximum(dinv * (a1 + y1_ref[...]) + b1_ref[...], 0.0)
        hw = jnp.dot(h, w2_ref[...], preferred_element_type=jnp.float32)
        y2p_ref[...] = jnp.broadcast_to(dinv * hw, (N, L))

    return pl.pallas_call(
        body, out_shape=jax.ShapeDtypeStruct((N, L), jnp.float32),
    )(a1p, y1, degp, b1, W2)


def _tc3(a2p, y2p, degp, b2):
    def body(a2p_ref, y2p_ref, degp_ref, b2_ref, o_ref):
        dinv = _dinv2(degp_ref[...])
        a2 = a2p_ref[0, :N, 0:1] + a2p_ref[1, :N, 0:1]
        y2 = y2p_ref[:, 0:1]
        o_ref[...] = jax.nn.sigmoid(dinv * (a2 + y2) + b2_ref[...])

    return pl.pallas_call(
        body, out_shape=jax.ShapeDtypeStruct((N, 1), jnp.float32),
    )(a2p, y2p, degp, b2)


def kernel(x, edge_index, W1, b1, W2, b2):
    e = edge_index.shape[1]
    grain = NW * CB * 2
    chn = 2 * (-(-e // grain))
    e_pad = chn * NW * CB
    ei = edge_index.astype(jnp.int32)
    src = jnp.concatenate([ei[0], jnp.zeros((e_pad - e,), jnp.int32)])
    dst = jnp.concatenate([ei[1], jnp.full((e_pad - e,), N, jnp.int32)])
    src3 = src.reshape(NW, chn, CB)
    dst3 = dst.reshape(NW, chn, CB)

    degp = _sc_deg(dst3)
    y1 = _tc1(x, W1, degp)
    a1p = _sc_agg(y1, src3, dst3, HID)
    y2p = _tc2(a1p, y1, degp, b1, W2)
    a2p = _sc_agg(y2p, src3, dst3, L)
    return _tc3(a2p, y2p, degp, b2)
